# baseline (speedup 1.0000x reference)
import functools

import jax
import jax.numpy as jnp
from jax import lax
from jax.experimental import pallas as pl
from jax.experimental.pallas import tpu as pltpu
from jax.experimental.pallas import tpu_sc as plsc

N = 10000
E = 320000
D_IN = 128
D_HID = 128
N_CLASSES = 64
EPS = 1e-5

NPAD = 10112
CHUNK = 128
NW = 32
CPW = 81
E_PAD = NW * CPW * CHUNK
ROWS_PER_SUB = NPAD // 16

_mesh = lambda: plsc.VectorSubcoreMesh(core_axis_name="c", subcore_axis_name="s")


def _zero_vmem_2d(ref, rows, d):
    def body(i, _):
        for j in range(d // 16):
            ref[i, pl.ds(j * 16, 16)] = jnp.zeros((16,), jnp.float32)
        return 0
    lax.fori_loop(0, rows, body, 0)


def _make_deg_kernel():
    @functools.partial(
        pl.kernel,
        out_type=jax.ShapeDtypeStruct((2 * NPAD,), jnp.float32),
        mesh=_mesh(),
        compiler_params=pltpu.CompilerParams(use_tc_tiling_on_sc=False),
        scratch_types=[
            pltpu.VMEM_SHARED((NPAD,), jnp.float32),
            pltpu.VMEM((CPW, CHUNK), jnp.int32),
            pltpu.VMEM((CHUNK,), jnp.float32),
            pltpu.VMEM((640,), jnp.float32),
        ],
    )
    def deg_kernel(dst_hbm, out_hbm, acc, dslab, ones_v, zv):
        c = lax.axis_index("c")
        s = lax.axis_index("s")
        w = c * 16 + s
        pltpu.sync_copy(dst_hbm.at[w], dslab)
        for j in range(CHUNK // 16):
            ones_v[pl.ds(j * 16, 16)] = jnp.ones((16,), jnp.float32)
        def zbody(i, _):
            zv[pl.ds(i * 16, 16)] = jnp.zeros((16,), jnp.float32)
            return 0
        lax.fori_loop(0, 640 // 16, zbody, 0)
        pltpu.sync_copy(zv.at[pl.ds(0, ROWS_PER_SUB)],
                        acc.at[pl.ds(s * ROWS_PER_SUB, ROWS_PER_SUB)])
        plsc.subcore_barrier()
        def body(k, _):
            pltpu.sync_copy(ones_v, acc.at[dslab.at[k]], add=True)
            return 0
        lax.fori_loop(0, CPW, body, 0)
        plsc.subcore_barrier()
        pltpu.sync_copy(
            acc.at[pl.ds(s * ROWS_PER_SUB, ROWS_PER_SUB)],
            out_hbm.at[pl.ds(c * NPAD + s * ROWS_PER_SUB, ROWS_PER_SUB)],
        )

    return deg_kernel


def _make_scatter_kernel(d):
    @functools.partial(
        pl.kernel,
        out_type=jax.ShapeDtypeStruct((2 * NPAD, d), jnp.float32),
        mesh=_mesh(),
        compiler_params=pltpu.CompilerParams(use_tc_tiling_on_sc=False),
        scratch_types=[
            pltpu.VMEM_SHARED((NPAD, d), jnp.float32),
            pltpu.VMEM((CHUNK, d), jnp.float32),
            pltpu.VMEM((CHUNK, d), jnp.float32),
            pltpu.VMEM((CHUNK, d), jnp.float32),
            pltpu.VMEM((CHUNK,), jnp.int32),
            pltpu.VMEM((CHUNK,), jnp.int32),
            pltpu.VMEM((CHUNK,), jnp.int32),
            pltpu.VMEM((CHUNK,), jnp.int32),
            pltpu.VMEM((CHUNK,), jnp.int32),
            pltpu.VMEM((CHUNK,), jnp.int32),
            pltpu.SemaphoreType.DMA,
            pltpu.SemaphoreType.DMA,
            pltpu.SemaphoreType.DMA,
            pltpu.SemaphoreType.DMA,
            pltpu.SemaphoreType.DMA,
            pltpu.SemaphoreType.DMA,
            pltpu.SemaphoreType.DMA,
            pltpu.SemaphoreType.DMA,
            pltpu.SemaphoreType.DMA,
            pltpu.SemaphoreType.DMA,
            pltpu.SemaphoreType.DMA,
            pltpu.SemaphoreType.DMA,
        ],
    )
    def scat_kernel(src_hbm, dst_hbm, h_hbm, out_hbm, acc,
                    rows0, rows1, rows2, sidx0, sidx1, sidx2,
                    didx0, didx1, didx2,
                    sem0, sem1, sem2, semi0, semi1, semi2,
                    semd0, semd1, semd2, sems0, sems1, sems2):
        c = lax.axis_index("c")
        s = lax.axis_index("s")
        w = c * 16 + s
        _zero_vmem_2d(rows0, CHUNK, d)
        for t in range(8):
            pltpu.sync_copy(rows0.at[pl.ds(0, ROWS_PER_SUB // 8)],
                            acc.at[pl.ds(s * ROWS_PER_SUB
                                         + t * (ROWS_PER_SUB // 8),
                                         ROWS_PER_SUB // 8)])
        plsc.subcore_barrier()
        bufs = ((rows0, sem0, sidx0, semi0, didx0, semd0, sems0),
                (rows1, sem1, sidx1, semi1, didx1, semd1, sems1),
                (rows2, sem2, sidx2, semi2, didx2, semd2, sems2))
        for b in range(3):
            pltpu.async_copy(src_hbm.at[w * CPW + b], bufs[b][2], bufs[b][3])
        for b in range(2):
            pltpu.async_copy(dst_hbm.at[w * CPW + b], bufs[b][4], bufs[b][5])
            pltpu.make_async_copy(src_hbm.at[w * CPW + b], bufs[b][2],
                                  bufs[b][3]).wait()
            pltpu.async_copy(h_hbm.at[bufs[b][2]], bufs[b][0], bufs[b][1])
        def body(j, _):
            for b in range(3):
                k = 3 * j + b
                rf, sf, sif, ssif, df, sdf, ssf = bufs[b]
                rn, sn, sin, ssin, dn, sdn, ssn = bufs[(b + 2) % 3]
                @pl.when(k >= 1)
                def _():
                    pltpu.make_async_copy(rn, acc.at[dn], ssn).wait()
                @pl.when(k + 2 < CPW)
                def _():
                    pltpu.make_async_copy(src_hbm.at[w * CPW + k + 2], sin,
                                          ssin).wait()
                    pltpu.async_copy(dst_hbm.at[w * CPW + k + 2], dn, sdn)
                    pltpu.async_copy(h_hbm.at[sin], rn, sn)
                pltpu.make_async_copy(h_hbm.at[sif], rf, sf).wait()
                @pl.when(k + 3 < CPW)
                def _():
                    pltpu.async_copy(src_hbm.at[w * CPW + k + 3], sif, ssif)
                pltpu.make_async_copy(dst_hbm.at[w * CPW + k], df, sdf).wait()
                pltpu.async_copy(rf, acc.at[df], ssf, add=True)
            return 0
        lax.fori_loop(0, CPW // 3, body, 0)
        lb = (CPW - 1) % 3
        pltpu.make_async_copy(bufs[lb][0], acc.at[bufs[lb][4]],
                              bufs[lb][6]).wait()
        plsc.subcore_barrier()
        pltpu.sync_copy(
            acc.at[pl.ds(s * ROWS_PER_SUB, ROWS_PER_SUB)],
            out_hbm.at[pl.ds(c * NPAD + s * ROWS_PER_SUB, ROWS_PER_SUB)],
        )

    return scat_kernel


_deg_call = _make_deg_kernel()
_scat128 = _make_scatter_kernel(D_HID)
_scat64 = _make_scatter_kernel(N_CLASSES)


def _col_broadcast(v):
    b = jnp.broadcast_to(v[None, :], (128, NPAD))
    ii = lax.broadcasted_iota(jnp.int32, (128, 128), 0)
    jj = lax.broadcasted_iota(jnp.int32, (128, 128), 1)
    eye = jnp.where(ii == jj, 1.0, 0.0)
    return lax.dot_general(b, eye, (((0,), (0,)), ((), ())),
                           preferred_element_type=jnp.float32)


def _tc0a_body(x_ref, w_ref, h_ref):
    h_ref[...] = jnp.dot(x_ref[...], w_ref[...],
                         preferred_element_type=jnp.float32)


_tc0a_call = pl.pallas_call(
    _tc0a_body,
    out_shape=jax.ShapeDtypeStruct((N, D_HID), jnp.float32),
)


def _tc0b_body(h_raw_ref, deg_ref, h_ref, dinvb_ref):
    deg = deg_ref[...]
    degs = deg[0:NPAD] + deg[NPAD:2 * NPAD]
    dinv = jnp.where(degs > 0, lax.rsqrt(degs), 0.0)
    dinvb = _col_broadcast(dinv)
    dinvb_ref[...] = dinvb
    h_ref[0:N, :] = dinvb[0:N, :] * h_raw_ref[...]
    h_ref[N:NPAD, :] = jnp.zeros((NPAD - N, D_HID), jnp.float32)


_tc0b_call = pl.pallas_call(
    _tc0b_body,
    out_shape=[
        jax.ShapeDtypeStruct((NPAD, D_HID), jnp.float32),
        jax.ShapeDtypeStruct((NPAD, 128), jnp.float32),
    ],
)


def _make_tc_mid(d_out):
    def body(acc_ref, dinvb_ref, b_ref, g_ref, be_ref, w_ref, out_ref):
        a = acc_ref[0:NPAD, :] + acc_ref[NPAD:2 * NPAD, :]
        h = a[0:N, :] * dinvb_ref[0:N, :] + b_ref[...]
        mu = jnp.mean(h, axis=0)
        xc = h - mu
        var = jnp.mean(xc * xc, axis=0)
        xh = xc * lax.rsqrt(var + EPS) * g_ref[...] + be_ref[...]
        r = jnp.maximum(xh, 0.0)
        o = jnp.dot(r, w_ref[...], preferred_element_type=jnp.float32)
        out_ref[0:N, :] = o * dinvb_ref[0:N, 0:d_out]
        out_ref[N:NPAD, :] = jnp.zeros((NPAD - N, d_out), jnp.float32)

    return pl.pallas_call(
        body,
        out_shape=jax.ShapeDtypeStruct((NPAD, d_out), jnp.float32),
    )


_tc_mid128 = _make_tc_mid(D_HID)
_tc_mid64 = _make_tc_mid(N_CLASSES)


def _tc3_body(acc_ref, dinvb_ref, b_ref, out_ref):
    a = acc_ref[0:NPAD, :] + acc_ref[NPAD:2 * NPAD, :]
    z = a[0:N, :] * dinvb_ref[0:N, 0:N_CLASSES] + b_ref[...]
    m = jnp.max(z, axis=1, keepdims=True)
    e = jnp.exp(z - m)
    lse = jnp.log(jnp.sum(e, axis=1, keepdims=True))
    out_ref[...] = z - m - lse


_tc3_call = pl.pallas_call(
    _tc3_body,
    out_shape=jax.ShapeDtypeStruct((N, N_CLASSES), jnp.float32),
)


def kernel(features, edge_index, W0, b0, gamma0, beta0, W1, b1, gamma1, beta1,
           W2, b2):
    loop = jnp.arange(N, dtype=jnp.int32)
    fill = N + (jnp.arange(E_PAD - E - N, dtype=jnp.int32) % (NPAD - N))
    srcp = jnp.concatenate([edge_index[0].astype(jnp.int32), loop, fill])
    dstp = jnp.concatenate([edge_index[1].astype(jnp.int32), loop, fill])
    src2 = srcp.reshape(NW * CPW, CHUNK)
    dst3 = dstp.reshape(NW, CPW, CHUNK)
    dst2 = dstp.reshape(NW * CPW, CHUNK)

    h0raw = _tc0a_call(features, W0)
    deg = _deg_call(dst3)
    h0, dinvb = _tc0b_call(h0raw, deg)
    acc0 = _scat128(src2, dst2, h0)
    h1 = _tc_mid128(acc0, dinvb, b0, gamma0, beta0, W1)
    acc1 = _scat128(src2, dst2, h1)
    h2 = _tc_mid64(acc1, dinvb, b1, gamma1, beta1, W2)
    acc2 = _scat64(src2, dst2, h2)
    return _tc3_call(acc2, dinvb, b2)


# speedup vs baseline: 1.0068x; 1.0068x over previous
import functools

import jax
import jax.numpy as jnp
from jax import lax
from jax.experimental import pallas as pl
from jax.experimental.pallas import tpu as pltpu
from jax.experimental.pallas import tpu_sc as plsc

N = 10000
E = 320000
D_IN = 128
D_HID = 128
N_CLASSES = 64
EPS = 1e-5

NPAD = 10112
CHUNK = 128
NW = 32
CPW = 81
E_PAD = NW * CPW * CHUNK
ROWS_PER_SUB = NPAD // 16

_mesh = lambda: plsc.VectorSubcoreMesh(core_axis_name="c", subcore_axis_name="s")


def _zero_vmem_2d(ref, rows, d):
    def body(i, _):
        for j in range(d // 16):
            ref[i, pl.ds(j * 16, 16)] = jnp.zeros((16,), jnp.float32)
        return 0
    lax.fori_loop(0, rows, body, 0)


def _make_deg_kernel():
    @functools.partial(
        pl.kernel,
        out_type=jax.ShapeDtypeStruct((2 * NPAD,), jnp.float32),
        mesh=_mesh(),
        compiler_params=pltpu.CompilerParams(use_tc_tiling_on_sc=False),
        scratch_types=[
            pltpu.VMEM_SHARED((NPAD,), jnp.float32),
            pltpu.VMEM((CPW, CHUNK), jnp.int32),
            pltpu.VMEM((CHUNK,), jnp.float32),
            pltpu.VMEM((640,), jnp.float32),
        ],
    )
    def deg_kernel(dst_hbm, out_hbm, acc, dslab, ones_v, zv):
        c = lax.axis_index("c")
        s = lax.axis_index("s")
        w = c * 16 + s
        pltpu.sync_copy(dst_hbm.at[w], dslab)
        for j in range(CHUNK // 16):
            ones_v[pl.ds(j * 16, 16)] = jnp.ones((16,), jnp.float32)
        def zbody(i, _):
            zv[pl.ds(i * 16, 16)] = jnp.zeros((16,), jnp.float32)
            return 0
        lax.fori_loop(0, 640 // 16, zbody, 0)
        pltpu.sync_copy(zv.at[pl.ds(0, ROWS_PER_SUB)],
                        acc.at[pl.ds(s * ROWS_PER_SUB, ROWS_PER_SUB)])
        plsc.subcore_barrier()
        def body(k, _):
            pltpu.sync_copy(ones_v, acc.at[dslab.at[k]], add=True)
            return 0
        lax.fori_loop(0, CPW, body, 0)
        plsc.subcore_barrier()
        pltpu.sync_copy(
            acc.at[pl.ds(s * ROWS_PER_SUB, ROWS_PER_SUB)],
            out_hbm.at[pl.ds(c * NPAD + s * ROWS_PER_SUB, ROWS_PER_SUB)],
        )

    return deg_kernel


def _make_scatter_kernel(d):
    @functools.partial(
        pl.kernel,
        out_type=jax.ShapeDtypeStruct((2 * NPAD, d), jnp.float32),
        mesh=_mesh(),
        compiler_params=pltpu.CompilerParams(use_tc_tiling_on_sc=False),
        scratch_types=[
            pltpu.VMEM_SHARED((NPAD, d), jnp.float32),
            pltpu.VMEM((CHUNK, d), jnp.float32),
            pltpu.VMEM((CHUNK, d), jnp.float32),
            pltpu.VMEM((CHUNK, d), jnp.float32),
            pltpu.VMEM((CHUNK,), jnp.int32),
            pltpu.VMEM((CHUNK,), jnp.int32),
            pltpu.VMEM((CHUNK,), jnp.int32),
            pltpu.VMEM((CHUNK,), jnp.int32),
            pltpu.VMEM((CHUNK,), jnp.int32),
            pltpu.VMEM((CHUNK,), jnp.int32),
            pltpu.SemaphoreType.DMA,
            pltpu.SemaphoreType.DMA,
            pltpu.SemaphoreType.DMA,
            pltpu.SemaphoreType.DMA,
            pltpu.SemaphoreType.DMA,
            pltpu.SemaphoreType.DMA,
            pltpu.SemaphoreType.DMA,
            pltpu.SemaphoreType.DMA,
            pltpu.SemaphoreType.DMA,
            pltpu.SemaphoreType.DMA,
            pltpu.SemaphoreType.DMA,
            pltpu.SemaphoreType.DMA,
        ],
    )
    def scat_kernel(src_hbm, dst_hbm, h_hbm, out_hbm, acc,
                    rows0, rows1, rows2, sidx0, sidx1, sidx2,
                    didx0, didx1, didx2,
                    sem0, sem1, sem2, semi0, semi1, semi2,
                    semd0, semd1, semd2, sems0, sems1, sems2):
        c = lax.axis_index("c")
        s = lax.axis_index("s")
        w = c * 16 + s
        _zero_vmem_2d(rows0, CHUNK, d)
        for t in range(8):
            pltpu.sync_copy(rows0.at[pl.ds(0, ROWS_PER_SUB // 8)],
                            acc.at[pl.ds(s * ROWS_PER_SUB
                                         + t * (ROWS_PER_SUB // 8),
                                         ROWS_PER_SUB // 8)])
        plsc.subcore_barrier()
        bufs = ((rows0, sem0, sidx0, semi0, didx0, semd0, sems0),
                (rows1, sem1, sidx1, semi1, didx1, semd1, sems1),
                (rows2, sem2, sidx2, semi2, didx2, semd2, sems2))
        for b in range(3):
            pltpu.async_copy(src_hbm.at[w * CPW + b], bufs[b][2], bufs[b][3])
        for b in range(2):
            pltpu.async_copy(dst_hbm.at[w * CPW + b], bufs[b][4], bufs[b][5])
            pltpu.make_async_copy(src_hbm.at[w * CPW + b], bufs[b][2],
                                  bufs[b][3]).wait()
            pltpu.async_copy(h_hbm.at[bufs[b][2]], bufs[b][0], bufs[b][1])
        def body(j, _):
            for b in range(3):
                k = 3 * j + b
                rf, sf, sif, ssif, df, sdf, ssf = bufs[b]
                rn, sn, sin, ssin, dn, sdn, ssn = bufs[(b + 2) % 3]
                @pl.when(k >= 1)
                def _():
                    pltpu.make_async_copy(rn, acc.at[dn], ssn).wait()
                @pl.when(k + 2 < CPW)
                def _():
                    pltpu.make_async_copy(src_hbm.at[w * CPW + k + 2], sin,
                                          ssin).wait()
                    pltpu.async_copy(dst_hbm.at[w * CPW + k + 2], dn, sdn)
                    pltpu.async_copy(h_hbm.at[sin], rn, sn)
                pltpu.make_async_copy(h_hbm.at[sif], rf, sf).wait()
                @pl.when(k + 3 < CPW)
                def _():
                    pltpu.async_copy(src_hbm.at[w * CPW + k + 3], sif, ssif)
                pltpu.make_async_copy(dst_hbm.at[w * CPW + k], df, sdf).wait()
                pltpu.async_copy(rf, acc.at[df], ssf, add=True)
            return 0
        lax.fori_loop(0, CPW // 3, body, 0)
        lb = (CPW - 1) % 3
        pltpu.make_async_copy(bufs[lb][0], acc.at[bufs[lb][4]],
                              bufs[lb][6]).wait()
        plsc.subcore_barrier()
        pltpu.sync_copy(
            acc.at[pl.ds(s * ROWS_PER_SUB, ROWS_PER_SUB)],
            out_hbm.at[pl.ds(c * NPAD + s * ROWS_PER_SUB, ROWS_PER_SUB)],
        )

    return scat_kernel


_deg_call = _make_deg_kernel()
_scat128 = _make_scatter_kernel(D_HID)
_scat64 = _make_scatter_kernel(N_CLASSES)


def _col_broadcast(v):
    b = jnp.broadcast_to(v[None, :], (128, NPAD))
    ii = lax.broadcasted_iota(jnp.int32, (128, 128), 0)
    jj = lax.broadcasted_iota(jnp.int32, (128, 128), 1)
    eye = jnp.where(ii == jj, 1.0, 0.0)
    return lax.dot_general(b, eye, (((0,), (0,)), ((), ())),
                           preferred_element_type=jnp.float32)


def _tc0_body(x_ref, w_ref, deg_ref, h_ref, dinvb_ref):
    deg = deg_ref[...]
    degs = deg[0:NPAD] + deg[NPAD:2 * NPAD]
    dinv = jnp.where(degs > 0, lax.rsqrt(degs), 0.0)
    dinvb = _col_broadcast(dinv)
    dinvb_ref[...] = dinvb
    h = jnp.dot(x_ref[...], w_ref[...], preferred_element_type=jnp.float32)
    h_ref[0:N, :] = dinvb[0:N, :] * h
    h_ref[N:NPAD, :] = jnp.zeros((NPAD - N, D_HID), jnp.float32)


_tc0_call = pl.pallas_call(
    _tc0_body,
    out_shape=[
        jax.ShapeDtypeStruct((NPAD, D_HID), jnp.float32),
        jax.ShapeDtypeStruct((NPAD, 128), jnp.float32),
    ],
)


def _make_tc_mid(d_out):
    def body(acc_ref, dinvb_ref, b_ref, g_ref, be_ref, w_ref, out_ref):
        a = acc_ref[0:NPAD, :] + acc_ref[NPAD:2 * NPAD, :]
        h = a[0:N, :] * dinvb_ref[0:N, :] + b_ref[...]
        mu = jnp.mean(h, axis=0)
        xc = h - mu
        var = jnp.mean(xc * xc, axis=0)
        xh = xc * lax.rsqrt(var + EPS) * g_ref[...] + be_ref[...]
        r = jnp.maximum(xh, 0.0)
        o = jnp.dot(r, w_ref[...], preferred_element_type=jnp.float32)
        out_ref[0:N, :] = o * dinvb_ref[0:N, 0:d_out]
        out_ref[N:NPAD, :] = jnp.zeros((NPAD - N, d_out), jnp.float32)

    return pl.pallas_call(
        body,
        out_shape=jax.ShapeDtypeStruct((NPAD, d_out), jnp.float32),
    )


_tc_mid128 = _make_tc_mid(D_HID)
_tc_mid64 = _make_tc_mid(N_CLASSES)


def _tc3_body(acc_ref, dinvb_ref, b_ref, out_ref):
    a = acc_ref[0:NPAD, :] + acc_ref[NPAD:2 * NPAD, :]
    z = a[0:N, :] * dinvb_ref[0:N, 0:N_CLASSES] + b_ref[...]
    m = jnp.max(z, axis=1, keepdims=True)
    e = jnp.exp(z - m)
    lse = jnp.log(jnp.sum(e, axis=1, keepdims=True))
    out_ref[...] = z - m - lse


_tc3_call = pl.pallas_call(
    _tc3_body,
    out_shape=jax.ShapeDtypeStruct((N, N_CLASSES), jnp.float32),
)


def kernel(features, edge_index, W0, b0, gamma0, beta0, W1, b1, gamma1, beta1,
           W2, b2):
    loop = jnp.arange(N, dtype=jnp.int32)
    fill = N + (jnp.arange(E_PAD - E - N, dtype=jnp.int32) % (NPAD - N))
    srcp = jnp.concatenate([edge_index[0].astype(jnp.int32), loop, fill])
    dstp = jnp.concatenate([edge_index[1].astype(jnp.int32), loop, fill])
    src2 = srcp.reshape(NW * CPW, CHUNK)
    dst3 = dstp.reshape(NW, CPW, CHUNK)
    dst2 = dstp.reshape(NW * CPW, CHUNK)

    deg = _deg_call(dst3)
    h0, dinvb = _tc0_call(features, W0, deg)
    acc0 = _scat128(src2, dst2, h0)
    h1 = _tc_mid128(acc0, dinvb, b0, gamma0, beta0, W1)
    acc1 = _scat128(src2, dst2, h1)
    h2 = _tc_mid64(acc1, dinvb, b1, gamma1, beta1, W2)
    acc2 = _scat64(src2, dst2, h2)
    return _tc3_call(acc2, dinvb, b2)
